# Initial kernel scaffold; baseline (speedup 1.0000x reference)
import jax
import jax.numpy as jnp
from jax.experimental import pallas as pl


def kernel(x, table):
    raise NotImplementedError("write your pallas kernel here")


# speedup vs baseline: 5.7166x; 5.7166x over previous
import functools

import jax
import jax.numpy as jnp
from jax import lax
from jax.experimental import pallas as pl
from jax.experimental.pallas import tpu as pltpu
from jax.experimental.pallas import tpu_sc as plsc

D_MODEL = 128
SCALE = float(D_MODEL) ** 0.5

_info = plsc.get_sparse_core_info()
_NC, _NS = _info.num_cores, _info.num_subcores
_NW = _NC * _NS

_B = 4096 * 200
_CHUNK = 128
_ROWS_PER_W = _B // _NW
_CHUNKS_PER_W = _ROWS_PER_W // _CHUNK


def _scale_body(t_ref, o_ref):
    o_ref[...] = t_ref[...] * SCALE


def _scale_table(table):
    rows = table.shape[0]
    blk = 2000
    return pl.pallas_call(
        _scale_body,
        out_shape=jax.ShapeDtypeStruct(table.shape, table.dtype),
        grid=(rows // blk,),
        in_specs=[pl.BlockSpec((blk, D_MODEL), lambda i: (i, 0))],
        out_specs=pl.BlockSpec((blk, D_MODEL), lambda i: (i, 0)),
    )(table)


@functools.partial(
    pl.kernel,
    mesh=plsc.VectorSubcoreMesh(core_axis_name="c", subcore_axis_name="s"),
    out_type=jax.ShapeDtypeStruct((_B, D_MODEL), jnp.float32),
    scratch_types=[
        pltpu.VMEM((_CHUNKS_PER_W, _CHUNK), jnp.int32),
        pltpu.VMEM((_CHUNK, D_MODEL), jnp.float32),
        pltpu.SemaphoreType.DMA,
    ],
)
def _sc_gather(table_hbm, idx_hbm, out_hbm, idx_v, rows_v, sem):
    wid = lax.axis_index("s") * _NC + lax.axis_index("c")
    ibase = wid * _CHUNKS_PER_W
    obase = wid * _ROWS_PER_W
    pltpu.sync_copy(idx_hbm.at[pl.ds(ibase, _CHUNKS_PER_W)], idx_v)

    def body(j, _):
        pltpu.async_copy(table_hbm.at[idx_v.at[j]], rows_v, sem).wait()
        pltpu.sync_copy(rows_v, out_hbm.at[pl.ds(obase + j * _CHUNK, _CHUNK)])
        return _

    lax.fori_loop(0, _CHUNKS_PER_W, body, None)


def kernel(x, table):
    table_scaled = _scale_table(table)
    idx = x.reshape(_B // _CHUNK, _CHUNK).astype(jnp.int32)
    out = _sc_gather(table_scaled, idx)
    return out.reshape(x.shape[0], x.shape[1], D_MODEL)


# speedup vs baseline: 7.9806x; 1.3960x over previous
import functools

import jax
import jax.numpy as jnp
from jax import lax
from jax.experimental import pallas as pl
from jax.experimental.pallas import tpu as pltpu
from jax.experimental.pallas import tpu_sc as plsc

D_MODEL = 128
SCALE = float(D_MODEL) ** 0.5

_info = plsc.get_sparse_core_info()
_NC, _NS = _info.num_cores, _info.num_subcores
_NW = _NC * _NS

_B = 4096 * 200
_CHUNK = 128
_ROWS_PER_W = _B // _NW
_CHUNKS_PER_W = _ROWS_PER_W // _CHUNK


def _scale_body(t_ref, o_ref):
    o_ref[...] = t_ref[...] * SCALE


def _scale_table(table):
    rows = table.shape[0]
    blk = 2000
    return pl.pallas_call(
        _scale_body,
        out_shape=jax.ShapeDtypeStruct(table.shape, table.dtype),
        grid=(rows // blk,),
        in_specs=[pl.BlockSpec((blk, D_MODEL), lambda i: (i, 0))],
        out_specs=pl.BlockSpec((blk, D_MODEL), lambda i: (i, 0)),
    )(table)


@functools.partial(
    pl.kernel,
    mesh=plsc.VectorSubcoreMesh(core_axis_name="c", subcore_axis_name="s"),
    out_type=jax.ShapeDtypeStruct((_B, D_MODEL), jnp.float32),
    scratch_types=[
        pltpu.VMEM((_CHUNKS_PER_W, _CHUNK), jnp.int32),
        pltpu.VMEM((_CHUNK, D_MODEL), jnp.float32),
        pltpu.VMEM((_CHUNK, D_MODEL), jnp.float32),
        pltpu.SemaphoreType.DMA,
        pltpu.SemaphoreType.DMA,
    ],
)
def _sc_gather(table_hbm, idx_hbm, out_hbm, idx_v, rows_a, rows_b, sem_a, sem_b):
    wid = lax.axis_index("s") * _NC + lax.axis_index("c")
    ibase = wid * _CHUNKS_PER_W
    obase = wid * _ROWS_PER_W
    pltpu.sync_copy(idx_hbm.at[pl.ds(ibase, _CHUNKS_PER_W)], idx_v)

    def gather(j, buf, sem):
        pltpu.make_async_copy(table_hbm.at[idx_v.at[j]], buf, sem).start()

    def gwait(buf, sem):
        pltpu.make_async_copy(table_hbm.at[idx_v.at[0]], buf, sem).wait()

    def scatter(j, buf):
        pltpu.sync_copy(buf, out_hbm.at[pl.ds(obase + j * _CHUNK, _CHUNK)])

    gather(0, rows_a, sem_a)
    npairs = _CHUNKS_PER_W // 2

    def body(i, carry):
        j0 = 2 * i
        gather(j0 + 1, rows_b, sem_b)
        gwait(rows_a, sem_a)
        scatter(j0, rows_a)

        @pl.when(i < npairs - 1)
        def _():
            gather(j0 + 2, rows_a, sem_a)

        gwait(rows_b, sem_b)
        scatter(j0 + 1, rows_b)
        return carry

    lax.fori_loop(0, npairs, body, 0)


def kernel(x, table):
    table_scaled = _scale_table(table)
    idx = x.reshape(_B // _CHUNK, _CHUNK).astype(jnp.int32)
    out = _sc_gather(table_scaled, idx)
    return out.reshape(x.shape[0], x.shape[1], D_MODEL)
